# baseline (speedup 1.0000x reference)
import functools

import jax
import jax.numpy as jnp
from jax import lax
from jax.experimental import pallas as pl

N = 10000
E = 320000
D = 128
K = 1000
KPAD = 1024
NPAD = 10240
BLK = 1280


def _ln(x, g, b):
    m = jnp.mean(x, axis=-1, keepdims=True)
    v = jnp.var(x, axis=-1, keepdims=True)
    return (x - m) / jnp.sqrt(v + 1e-5) * g + b


def _seg_softmax_agg(e, mask, src, dst, h, n):
    maskf = mask.astype(jnp.float32)
    e = jnp.where(mask, e, jnp.float32(-1e9))
    m = jax.ops.segment_max(e, dst, num_segments=n)
    m = jnp.where(jnp.isfinite(m), m, jnp.float32(0.0))
    ex = jnp.exp(e - m[dst]) * maskf
    denom = jax.ops.segment_sum(ex, dst, num_segments=n)
    alpha = ex / jnp.maximum(denom[dst], 1e-16)
    return jax.ops.segment_sum(alpha[:, None] * h[src], dst, num_segments=n)


def _conv(x, edge_index, node_type, W, b, asi, adi, ase, ade):
    h = x @ W
    src = edge_index[0]
    dst = edge_index[1]
    inter = node_type[src] != node_type[dst]
    n = x.shape[0]
    e_intra = jax.nn.leaky_relu((h @ asi)[src] + (h @ adi)[dst], 0.2)
    e_inter = jax.nn.leaky_relu((h @ ase)[src] + (h @ ade)[dst], 0.2)
    out = _seg_softmax_agg(e_intra, jnp.logical_not(inter), src, dst, h, n)
    out = out + _seg_softmax_agg(e_inter, inter, src, dst, h, n)
    return out + b


def _pool_body(h_ref, sc_ref, cl_ref, out_ref, cnt_ref):
    i = pl.program_id(0)

    @pl.when(i == 0)
    def _init():
        out_ref[...] = jnp.zeros_like(out_ref)
        cnt_ref[...] = jnp.zeros_like(cnt_ref)

    hv = h_ref[...]
    sc = sc_ref[...]
    cl = cl_ref[...]
    onehot = (cl == lax.broadcasted_iota(jnp.int32, (BLK, KPAD), 1)).astype(jnp.float32)
    wrow = hv * sc
    out_ref[...] += lax.dot_general(onehot, wrow, (((0,), (0,)), ((), ())),
                                    preferred_element_type=jnp.float32)
    cnt_ref[...] += lax.dot_general(onehot, jnp.ones((BLK, 128), jnp.float32),
                                    (((0,), (0,)), ((), ())),
                                    preferred_element_type=jnp.float32)

    @pl.when(i == pl.num_programs(0) - 1)
    def _fin():
        out_ref[...] = out_ref[...] / jnp.maximum(cnt_ref[...][:, :1], 1.0)


@functools.partial(jax.jit, static_argnums=())
def _pool_pallas(h, score, cluster):
    hp = jnp.zeros((NPAD, D), jnp.float32).at[:N].set(h)
    sp = jnp.zeros((NPAD, 1), jnp.float32).at[:N, 0].set(score)
    cp = jnp.full((NPAD, 1), KPAD - 1, jnp.int32).at[:N, 0].set(cluster)
    grid = NPAD // BLK
    out, _ = pl.pallas_call(
        _pool_body,
        grid=(grid,),
        in_specs=[
            pl.BlockSpec((BLK, D), lambda i: (i, 0)),
            pl.BlockSpec((BLK, 1), lambda i: (i, 0)),
            pl.BlockSpec((BLK, 1), lambda i: (i, 0)),
        ],
        out_specs=[
            pl.BlockSpec((KPAD, D), lambda i: (0, 0)),
            pl.BlockSpec((KPAD, 128), lambda i: (0, 0)),
        ],
        out_shape=[
            jax.ShapeDtypeStruct((KPAD, D), jnp.float32),
            jax.ShapeDtypeStruct((KPAD, 128), jnp.float32),
        ],
    )(hp, sp, cp)
    return out[:K]


def kernel(x, edge_index, node_type, data_id, tree, x_y_index, batch,
           ln_gamma, ln_beta, W_conv, bias_conv, att_src_intra, att_dst_intra,
           att_src_inter, att_dst_inter, pool_w):
    xy = x_y_index * 2.0 - 1.0
    h = _ln(x, ln_gamma, ln_beta)
    h = _conv(h, edge_index, node_type, W_conv, bias_conv,
              att_src_intra, att_dst_intra, att_src_inter, att_dst_inter)
    h = _ln(h, ln_gamma, ln_beta)

    w = pool_w
    score = jnp.tanh(h @ (w / (jnp.linalg.norm(w) + 1e-12)))
    _, kept = jax.lax.top_k(score, K)
    centers = xy[kept]
    d2 = (jnp.sum(xy * xy, axis=-1)[:, None]
          + jnp.sum(centers * centers, axis=-1)[None, :]
          - 2.0 * (xy @ centers.T))
    cluster = jnp.argmin(d2, axis=1)

    new_x = _pool_pallas(h, score, cluster)
    new_edge = cluster[edge_index]
    nt = node_type[kept]
    tr = tree[kept]
    return (new_x, new_edge.astype(jnp.int32), nt, tr)


# speedup vs baseline: 1.0220x; 1.0220x over previous
import functools

import jax
import jax.numpy as jnp
from jax import lax
from jax.experimental import pallas as pl

N = 10000
E = 320000
D = 128
K = 1000
KPAD = 1024
NPAD = 10240
BLK = 1280


def _ln(x, g, b):
    m = jnp.mean(x, axis=-1, keepdims=True)
    v = jnp.var(x, axis=-1, keepdims=True)
    return (x - m) / jnp.sqrt(v + 1e-5) * g + b


def _seg_softmax_agg(e, mask, src, dst, ds, pi, h, n):
    maskf = mask.astype(jnp.float32)
    e = jnp.where(mask, e, jnp.float32(-1e9))
    m = jax.ops.segment_max(e[pi], ds, num_segments=n, indices_are_sorted=True)
    m = jnp.where(jnp.isfinite(m), m, jnp.float32(0.0))
    ex = jnp.exp(e - m[dst]) * maskf
    denom = jax.ops.segment_sum(ex, dst, num_segments=n)
    alpha = ex / jnp.maximum(denom[dst], 1e-16)
    return jax.ops.segment_sum(alpha[:, None] * h[src], dst, num_segments=n)


def _conv(x, edge_index, node_type, W, b, asi, adi, ase, ade):
    h = x @ W
    src = edge_index[0]
    dst = edge_index[1]
    ds, pi = lax.sort((dst, lax.iota(jnp.int32, dst.shape[0])), num_keys=1)
    inter = node_type[src] != node_type[dst]
    n = x.shape[0]
    e_intra = jax.nn.leaky_relu((h @ asi)[src] + (h @ adi)[dst], 0.2)
    e_inter = jax.nn.leaky_relu((h @ ase)[src] + (h @ ade)[dst], 0.2)
    out = _seg_softmax_agg(e_intra, jnp.logical_not(inter), src, dst, ds, pi, h, n)
    out = out + _seg_softmax_agg(e_inter, inter, src, dst, ds, pi, h, n)
    return out + b


def _pool_body(h_ref, sc_ref, cl_ref, out_ref, cnt_ref):
    i = pl.program_id(0)

    @pl.when(i == 0)
    def _init():
        out_ref[...] = jnp.zeros_like(out_ref)
        cnt_ref[...] = jnp.zeros_like(cnt_ref)

    hv = h_ref[...]
    sc = sc_ref[...]
    cl = cl_ref[...]
    onehot = (cl == lax.broadcasted_iota(jnp.int32, (BLK, KPAD), 1)).astype(jnp.float32)
    wrow = hv * sc
    out_ref[...] += lax.dot_general(onehot, wrow, (((0,), (0,)), ((), ())),
                                    preferred_element_type=jnp.float32)
    cnt_ref[...] += lax.dot_general(onehot, jnp.ones((BLK, 128), jnp.float32),
                                    (((0,), (0,)), ((), ())),
                                    preferred_element_type=jnp.float32)

    @pl.when(i == pl.num_programs(0) - 1)
    def _fin():
        out_ref[...] = out_ref[...] / jnp.maximum(cnt_ref[...][:, :1], 1.0)


@functools.partial(jax.jit, static_argnums=())
def _pool_pallas(h, score, cluster):
    hp = jnp.zeros((NPAD, D), jnp.float32).at[:N].set(h)
    sp = jnp.zeros((NPAD, 1), jnp.float32).at[:N, 0].set(score)
    cp = jnp.full((NPAD, 1), KPAD - 1, jnp.int32).at[:N, 0].set(cluster)
    grid = NPAD // BLK
    out, _ = pl.pallas_call(
        _pool_body,
        grid=(grid,),
        in_specs=[
            pl.BlockSpec((BLK, D), lambda i: (i, 0)),
            pl.BlockSpec((BLK, 1), lambda i: (i, 0)),
            pl.BlockSpec((BLK, 1), lambda i: (i, 0)),
        ],
        out_specs=[
            pl.BlockSpec((KPAD, D), lambda i: (0, 0)),
            pl.BlockSpec((KPAD, 128), lambda i: (0, 0)),
        ],
        out_shape=[
            jax.ShapeDtypeStruct((KPAD, D), jnp.float32),
            jax.ShapeDtypeStruct((KPAD, 128), jnp.float32),
        ],
    )(hp, sp, cp)
    return out[:K]


def kernel(x, edge_index, node_type, data_id, tree, x_y_index, batch,
           ln_gamma, ln_beta, W_conv, bias_conv, att_src_intra, att_dst_intra,
           att_src_inter, att_dst_inter, pool_w):
    xy = x_y_index * 2.0 - 1.0
    h = _ln(x, ln_gamma, ln_beta)
    h = _conv(h, edge_index, node_type, W_conv, bias_conv,
              att_src_intra, att_dst_intra, att_src_inter, att_dst_inter)
    h = _ln(h, ln_gamma, ln_beta)

    w = pool_w
    score = jnp.tanh(h @ (w / (jnp.linalg.norm(w) + 1e-12)))
    _, kept = jax.lax.top_k(score, K)
    centers = xy[kept]
    d2 = (jnp.sum(xy * xy, axis=-1)[:, None]
          + jnp.sum(centers * centers, axis=-1)[None, :]
          - 2.0 * (xy @ centers.T))
    cluster = jnp.argmin(d2, axis=1)

    new_x = _pool_pallas(h, score, cluster)
    new_edge = cluster[edge_index]
    nt = node_type[kept]
    tr = tree[kept]
    return (new_x, new_edge.astype(jnp.int32), nt, tr)


# speedup vs baseline: 7.5887x; 7.4256x over previous
import functools

import jax
import jax.numpy as jnp
from jax import lax
from jax.experimental import pallas as pl
from jax.experimental.pallas import tpu as pltpu
from jax.experimental.pallas import tpu_sc as plsc

N = 10000
E = 320000
D = 128
K = 1000
KPAD = 1024
NPAD = 10240
BLK = 1280


_NW = 32


def _sc_gather(table, idx):
    B = idx.shape[0]
    bpw = B // _NW
    mesh = plsc.VectorSubcoreMesh(core_axis_name="c", subcore_axis_name="s")

    @functools.partial(
        pl.kernel, mesh=mesh,
        out_type=jax.ShapeDtypeStruct((B,), table.dtype),
        scratch_types=[
            pltpu.VMEM((bpw,), jnp.int32),
            pltpu.VMEM((bpw,), table.dtype),
            pltpu.SemaphoreType.DMA,
        ],
    )
    def k(table_hbm, idx_hbm, out_hbm, idx_v, rows_v, sem):
        wid = lax.axis_index("s") * 2 + lax.axis_index("c")
        base = wid * bpw
        pltpu.sync_copy(idx_hbm.at[pl.ds(base, bpw)], idx_v)
        pltpu.async_copy(table_hbm.at[idx_v], rows_v, sem).wait()
        pltpu.sync_copy(rows_v, out_hbm.at[pl.ds(base, bpw)])

    return k(table, idx)


def _ln(x, g, b):
    m = jnp.mean(x, axis=-1, keepdims=True)
    v = jnp.var(x, axis=-1, keepdims=True)
    return (x - m) / jnp.sqrt(v + 1e-5) * g + b


def _seg_softmax_agg(e, mask, src, dst, ds, pi, h, n):
    maskf = mask.astype(jnp.float32)
    e = jnp.where(mask, e, jnp.float32(-1e9))
    m = jax.ops.segment_max(_sc_gather(e, pi), ds, num_segments=n,
                            indices_are_sorted=True)
    m = jnp.where(jnp.isfinite(m), m, jnp.float32(0.0))
    ex = jnp.exp(e - _sc_gather(m, dst)) * maskf
    denom = jax.ops.segment_sum(ex, dst, num_segments=n)
    alpha = ex / jnp.maximum(_sc_gather(denom, dst), 1e-16)
    return jax.ops.segment_sum(alpha[:, None] * h[src], dst, num_segments=n)


def _conv(x, edge_index, node_type, W, b, asi, adi, ase, ade):
    h = x @ W
    src = edge_index[0]
    dst = edge_index[1]
    ds, pi = lax.sort((dst, lax.iota(jnp.int32, dst.shape[0])), num_keys=1)
    inter = _sc_gather(node_type, src) != _sc_gather(node_type, dst)
    n = x.shape[0]
    e_intra = jax.nn.leaky_relu(_sc_gather(h @ asi, src)
                                + _sc_gather(h @ adi, dst), 0.2)
    e_inter = jax.nn.leaky_relu(_sc_gather(h @ ase, src)
                                + _sc_gather(h @ ade, dst), 0.2)
    out = _seg_softmax_agg(e_intra, jnp.logical_not(inter), src, dst, ds, pi, h, n)
    out = out + _seg_softmax_agg(e_inter, inter, src, dst, ds, pi, h, n)
    return out + b


def _pool_body(h_ref, sc_ref, cl_ref, out_ref, cnt_ref):
    i = pl.program_id(0)

    @pl.when(i == 0)
    def _init():
        out_ref[...] = jnp.zeros_like(out_ref)
        cnt_ref[...] = jnp.zeros_like(cnt_ref)

    hv = h_ref[...]
    sc = sc_ref[...]
    cl = cl_ref[...]
    onehot = (cl == lax.broadcasted_iota(jnp.int32, (BLK, KPAD), 1)).astype(jnp.float32)
    wrow = hv * sc
    out_ref[...] += lax.dot_general(onehot, wrow, (((0,), (0,)), ((), ())),
                                    preferred_element_type=jnp.float32)
    cnt_ref[...] += lax.dot_general(onehot, jnp.ones((BLK, 128), jnp.float32),
                                    (((0,), (0,)), ((), ())),
                                    preferred_element_type=jnp.float32)

    @pl.when(i == pl.num_programs(0) - 1)
    def _fin():
        out_ref[...] = out_ref[...] / jnp.maximum(cnt_ref[...][:, :1], 1.0)


@functools.partial(jax.jit, static_argnums=())
def _pool_pallas(h, score, cluster):
    hp = jnp.zeros((NPAD, D), jnp.float32).at[:N].set(h)
    sp = jnp.zeros((NPAD, 1), jnp.float32).at[:N, 0].set(score)
    cp = jnp.full((NPAD, 1), KPAD - 1, jnp.int32).at[:N, 0].set(cluster)
    grid = NPAD // BLK
    out, _ = pl.pallas_call(
        _pool_body,
        grid=(grid,),
        in_specs=[
            pl.BlockSpec((BLK, D), lambda i: (i, 0)),
            pl.BlockSpec((BLK, 1), lambda i: (i, 0)),
            pl.BlockSpec((BLK, 1), lambda i: (i, 0)),
        ],
        out_specs=[
            pl.BlockSpec((KPAD, D), lambda i: (0, 0)),
            pl.BlockSpec((KPAD, 128), lambda i: (0, 0)),
        ],
        out_shape=[
            jax.ShapeDtypeStruct((KPAD, D), jnp.float32),
            jax.ShapeDtypeStruct((KPAD, 128), jnp.float32),
        ],
    )(hp, sp, cp)
    return out[:K]


def kernel(x, edge_index, node_type, data_id, tree, x_y_index, batch,
           ln_gamma, ln_beta, W_conv, bias_conv, att_src_intra, att_dst_intra,
           att_src_inter, att_dst_inter, pool_w):
    xy = x_y_index * 2.0 - 1.0
    h = _ln(x, ln_gamma, ln_beta)
    h = _conv(h, edge_index, node_type, W_conv, bias_conv,
              att_src_intra, att_dst_intra, att_src_inter, att_dst_inter)
    h = _ln(h, ln_gamma, ln_beta)

    w = pool_w
    score = jnp.tanh(h @ (w / (jnp.linalg.norm(w) + 1e-12)))
    _, kept = jax.lax.top_k(score, K)
    centers = xy[kept]
    d2 = (jnp.sum(xy * xy, axis=-1)[:, None]
          + jnp.sum(centers * centers, axis=-1)[None, :]
          - 2.0 * (xy @ centers.T))
    cluster = jnp.argmin(d2, axis=1)

    new_x = _pool_pallas(h, score, cluster)
    new_edge = _sc_gather(cluster.astype(jnp.int32),
                          edge_index.reshape(2 * E)).reshape(2, E)
    nt = node_type[kept]
    tr = tree[kept]
    return (new_x, new_edge.astype(jnp.int32), nt, tr)


# speedup vs baseline: 9.0229x; 1.1890x over previous
import functools

import jax
import jax.numpy as jnp
from jax import lax
from jax.experimental import pallas as pl
from jax.experimental.pallas import tpu as pltpu
from jax.experimental.pallas import tpu_sc as plsc

N = 10000
E = 320000
D = 128
K = 1000
KPAD = 1024
NPAD = 10240
BLK = 1280


_NW = 32


def _sc_gather(table, idx):
    B = idx.shape[0]
    bpw = B // _NW
    mesh = plsc.VectorSubcoreMesh(core_axis_name="c", subcore_axis_name="s")

    @functools.partial(
        pl.kernel, mesh=mesh,
        out_type=jax.ShapeDtypeStruct((B,), table.dtype),
        scratch_types=[
            pltpu.VMEM((bpw,), jnp.int32),
            pltpu.VMEM((bpw,), table.dtype),
            pltpu.SemaphoreType.DMA,
        ],
    )
    def k(table_hbm, idx_hbm, out_hbm, idx_v, rows_v, sem):
        wid = lax.axis_index("s") * 2 + lax.axis_index("c")
        base = wid * bpw
        pltpu.sync_copy(idx_hbm.at[pl.ds(base, bpw)], idx_v)
        pltpu.async_copy(table_hbm.at[idx_v], rows_v, sem).wait()
        pltpu.sync_copy(rows_v, out_hbm.at[pl.ds(base, bpw)])

    return k(table, idx)


def _sc_gather_rows(table, idx):
    B = idx.shape[0]
    bpw = B // _NW
    CH = 400
    nch = bpw // CH
    mesh = plsc.VectorSubcoreMesh(core_axis_name="c", subcore_axis_name="s")

    @functools.partial(
        pl.kernel, mesh=mesh,
        out_type=jax.ShapeDtypeStruct((B, D), table.dtype),
        scratch_types=[
            pltpu.VMEM((bpw,), jnp.int32),
            pltpu.VMEM((CH, D), table.dtype),
            pltpu.SemaphoreType.DMA,
        ],
    )
    def k(table_hbm, idx_hbm, out_hbm, idx_v, rows_v, sem):
        wid = lax.axis_index("s") * 2 + lax.axis_index("c")
        base = wid * bpw
        pltpu.sync_copy(idx_hbm.at[pl.ds(base, bpw)], idx_v)
        for j in range(nch):
            pltpu.async_copy(table_hbm.at[idx_v.at[pl.ds(j * CH, CH)]],
                             rows_v, sem).wait()
            pltpu.sync_copy(rows_v, out_hbm.at[pl.ds(base + j * CH, CH)])

    return k(table, idx)


def _ln(x, g, b):
    m = jnp.mean(x, axis=-1, keepdims=True)
    v = jnp.var(x, axis=-1, keepdims=True)
    return (x - m) / jnp.sqrt(v + 1e-5) * g + b


def _seg_softmax_agg(e, mask, src, dst, ds, pi, h_src, n):
    maskf = mask.astype(jnp.float32)
    e = jnp.where(mask, e, jnp.float32(-1e9))
    m = jax.ops.segment_max(_sc_gather(e, pi), ds, num_segments=n,
                            indices_are_sorted=True)
    m = jnp.where(jnp.isfinite(m), m, jnp.float32(0.0))
    ex = jnp.exp(e - _sc_gather(m, dst)) * maskf
    denom = jax.ops.segment_sum(ex, dst, num_segments=n)
    alpha = ex / jnp.maximum(_sc_gather(denom, dst), 1e-16)
    return jax.ops.segment_sum(alpha[:, None] * h_src, dst, num_segments=n)


def _conv(x, edge_index, node_type, W, b, asi, adi, ase, ade):
    h = x @ W
    src = edge_index[0]
    dst = edge_index[1]
    ds, pi = lax.sort((dst, lax.iota(jnp.int32, dst.shape[0])), num_keys=1)
    inter = _sc_gather(node_type, src) != _sc_gather(node_type, dst)
    n = x.shape[0]
    e_intra = jax.nn.leaky_relu(_sc_gather(h @ asi, src)
                                + _sc_gather(h @ adi, dst), 0.2)
    e_inter = jax.nn.leaky_relu(_sc_gather(h @ ase, src)
                                + _sc_gather(h @ ade, dst), 0.2)
    h_src = _sc_gather_rows(h, src)
    out = _seg_softmax_agg(e_intra, jnp.logical_not(inter), src, dst, ds, pi, h_src, n)
    out = out + _seg_softmax_agg(e_inter, inter, src, dst, ds, pi, h_src, n)
    return out + b


def _pool_body(h_ref, sc_ref, cl_ref, out_ref, cnt_ref):
    i = pl.program_id(0)

    @pl.when(i == 0)
    def _init():
        out_ref[...] = jnp.zeros_like(out_ref)
        cnt_ref[...] = jnp.zeros_like(cnt_ref)

    hv = h_ref[...]
    sc = sc_ref[...]
    cl = cl_ref[...]
    onehot = (cl == lax.broadcasted_iota(jnp.int32, (BLK, KPAD), 1)).astype(jnp.float32)
    wrow = hv * sc
    out_ref[...] += lax.dot_general(onehot, wrow, (((0,), (0,)), ((), ())),
                                    preferred_element_type=jnp.float32)
    cnt_ref[...] += lax.dot_general(onehot, jnp.ones((BLK, 128), jnp.float32),
                                    (((0,), (0,)), ((), ())),
                                    preferred_element_type=jnp.float32)

    @pl.when(i == pl.num_programs(0) - 1)
    def _fin():
        out_ref[...] = out_ref[...] / jnp.maximum(cnt_ref[...][:, :1], 1.0)


@functools.partial(jax.jit, static_argnums=())
def _pool_pallas(h, score, cluster):
    hp = jnp.zeros((NPAD, D), jnp.float32).at[:N].set(h)
    sp = jnp.zeros((NPAD, 1), jnp.float32).at[:N, 0].set(score)
    cp = jnp.full((NPAD, 1), KPAD - 1, jnp.int32).at[:N, 0].set(cluster)
    grid = NPAD // BLK
    out, _ = pl.pallas_call(
        _pool_body,
        grid=(grid,),
        in_specs=[
            pl.BlockSpec((BLK, D), lambda i: (i, 0)),
            pl.BlockSpec((BLK, 1), lambda i: (i, 0)),
            pl.BlockSpec((BLK, 1), lambda i: (i, 0)),
        ],
        out_specs=[
            pl.BlockSpec((KPAD, D), lambda i: (0, 0)),
            pl.BlockSpec((KPAD, 128), lambda i: (0, 0)),
        ],
        out_shape=[
            jax.ShapeDtypeStruct((KPAD, D), jnp.float32),
            jax.ShapeDtypeStruct((KPAD, 128), jnp.float32),
        ],
    )(hp, sp, cp)
    return out[:K]


def kernel(x, edge_index, node_type, data_id, tree, x_y_index, batch,
           ln_gamma, ln_beta, W_conv, bias_conv, att_src_intra, att_dst_intra,
           att_src_inter, att_dst_inter, pool_w):
    xy = x_y_index * 2.0 - 1.0
    h = _ln(x, ln_gamma, ln_beta)
    h = _conv(h, edge_index, node_type, W_conv, bias_conv,
              att_src_intra, att_dst_intra, att_src_inter, att_dst_inter)
    h = _ln(h, ln_gamma, ln_beta)

    w = pool_w
    score = jnp.tanh(h @ (w / (jnp.linalg.norm(w) + 1e-12)))
    _, kept = jax.lax.top_k(score, K)
    centers = xy[kept]
    d2 = (jnp.sum(xy * xy, axis=-1)[:, None]
          + jnp.sum(centers * centers, axis=-1)[None, :]
          - 2.0 * (xy @ centers.T))
    cluster = jnp.argmin(d2, axis=1)

    new_x = _pool_pallas(h, score, cluster)
    new_edge = _sc_gather(cluster.astype(jnp.int32),
                          edge_index.reshape(2 * E)).reshape(2, E)
    nt = node_type[kept]
    tr = tree[kept]
    return (new_x, new_edge.astype(jnp.int32), nt, tr)


# speedup vs baseline: 9.2071x; 1.0204x over previous
import functools

import jax
import jax.numpy as jnp
from jax import lax
from jax.experimental import pallas as pl
from jax.experimental.pallas import tpu as pltpu
from jax.experimental.pallas import tpu_sc as plsc

N = 10000
E = 320000
D = 128
K = 1000
KPAD = 1024
NPAD = 10240
BLK = 1280


_NW = 32


def _sc_gather(table, idx):
    B = idx.shape[0]
    bpw = B // _NW
    mesh = plsc.VectorSubcoreMesh(core_axis_name="c", subcore_axis_name="s")

    @functools.partial(
        pl.kernel, mesh=mesh,
        out_type=jax.ShapeDtypeStruct((B,), table.dtype),
        scratch_types=[
            pltpu.VMEM((bpw,), jnp.int32),
            pltpu.VMEM((bpw,), table.dtype),
            pltpu.SemaphoreType.DMA,
        ],
    )
    def k(table_hbm, idx_hbm, out_hbm, idx_v, rows_v, sem):
        wid = lax.axis_index("s") * 2 + lax.axis_index("c")
        base = wid * bpw
        pltpu.sync_copy(idx_hbm.at[pl.ds(base, bpw)], idx_v)
        pltpu.async_copy(table_hbm.at[idx_v], rows_v, sem).wait()
        pltpu.sync_copy(rows_v, out_hbm.at[pl.ds(base, bpw)])

    return k(table, idx)


def _sc_gather_rows(table, idx):
    B = idx.shape[0]
    bpw = B // _NW
    CH = 400
    nch = bpw // CH
    mesh = plsc.VectorSubcoreMesh(core_axis_name="c", subcore_axis_name="s")

    @functools.partial(
        pl.kernel, mesh=mesh,
        out_type=jax.ShapeDtypeStruct((B, D), table.dtype),
        scratch_types=[
            pltpu.VMEM((bpw,), jnp.int32),
            pltpu.VMEM((CH, D), table.dtype),
            pltpu.SemaphoreType.DMA,
        ],
    )
    def k(table_hbm, idx_hbm, out_hbm, idx_v, rows_v, sem):
        wid = lax.axis_index("s") * 2 + lax.axis_index("c")
        base = wid * bpw
        pltpu.sync_copy(idx_hbm.at[pl.ds(base, bpw)], idx_v)
        for j in range(nch):
            pltpu.async_copy(table_hbm.at[idx_v.at[pl.ds(j * CH, CH)]],
                             rows_v, sem).wait()
            pltpu.sync_copy(rows_v, out_hbm.at[pl.ds(base + j * CH, CH)])

    return k(table, idx)


def _sc_gather6(t_si, t_di, t_se, t_de, t_nt, src, dst):
    B = src.shape[0]
    bpw = B // _NW
    mesh = plsc.VectorSubcoreMesh(core_axis_name="c", subcore_axis_name="s")

    @functools.partial(
        pl.kernel, mesh=mesh,
        out_type=[jax.ShapeDtypeStruct((B,), jnp.float32)] * 4
                 + [jax.ShapeDtypeStruct((B,), jnp.int32)] * 2,
        scratch_types=[
            pltpu.VMEM((bpw,), jnp.int32),
            pltpu.VMEM((bpw,), jnp.int32),
            pltpu.VMEM((bpw,), jnp.float32),
            pltpu.VMEM((bpw,), jnp.int32),
            pltpu.SemaphoreType.DMA,
        ],
    )
    def k(si_h, di_h, se_h, de_h, nt_h, src_h, dst_h,
          o_si, o_di, o_se, o_de, o_nts, o_ntd,
          src_v, dst_v, buf_f, buf_i, sem):
        wid = lax.axis_index("s") * 2 + lax.axis_index("c")
        base = wid * bpw
        pltpu.sync_copy(src_h.at[pl.ds(base, bpw)], src_v)
        pltpu.sync_copy(dst_h.at[pl.ds(base, bpw)], dst_v)
        for tab, idx_v, buf, out in ((si_h, src_v, buf_f, o_si),
                                     (di_h, dst_v, buf_f, o_di),
                                     (se_h, src_v, buf_f, o_se),
                                     (de_h, dst_v, buf_f, o_de),
                                     (nt_h, src_v, buf_i, o_nts),
                                     (nt_h, dst_v, buf_i, o_ntd)):
            pltpu.async_copy(tab.at[idx_v], buf, sem).wait()
            pltpu.sync_copy(buf, out.at[pl.ds(base, bpw)])

    return k(t_si, t_di, t_se, t_de, t_nt, src, dst)


def _ln(x, g, b):
    m = jnp.mean(x, axis=-1, keepdims=True)
    v = jnp.var(x, axis=-1, keepdims=True)
    return (x - m) / jnp.sqrt(v + 1e-5) * g + b


def _seg_softmax_agg(e, mask, src, dst, ds, pi, h_src, n):
    maskf = mask.astype(jnp.float32)
    e = jnp.where(mask, e, jnp.float32(-1e9))
    m = jax.ops.segment_max(_sc_gather(e, pi), ds, num_segments=n,
                            indices_are_sorted=True)
    m = jnp.where(jnp.isfinite(m), m, jnp.float32(0.0))
    ex = jnp.exp(e - _sc_gather(m, dst)) * maskf
    denom = jax.ops.segment_sum(ex, dst, num_segments=n)
    alpha = ex / jnp.maximum(_sc_gather(denom, dst), 1e-16)
    return jax.ops.segment_sum(alpha[:, None] * h_src, dst, num_segments=n)


def _conv(x, edge_index, node_type, W, b, asi, adi, ase, ade):
    h = x @ W
    src = edge_index[0]
    dst = edge_index[1]
    ds, pi = lax.sort((dst, lax.iota(jnp.int32, dst.shape[0])), num_keys=1)
    n = x.shape[0]
    g_si, g_di, g_se, g_de, g_nts, g_ntd = _sc_gather6(
        h @ asi, h @ adi, h @ ase, h @ ade, node_type, src, dst)
    inter = g_nts != g_ntd
    e_intra = jax.nn.leaky_relu(g_si + g_di, 0.2)
    e_inter = jax.nn.leaky_relu(g_se + g_de, 0.2)
    h_src = _sc_gather_rows(h, src)
    out = _seg_softmax_agg(e_intra, jnp.logical_not(inter), src, dst, ds, pi, h_src, n)
    out = out + _seg_softmax_agg(e_inter, inter, src, dst, ds, pi, h_src, n)
    return out + b


def _pool_body(h_ref, sc_ref, cl_ref, out_ref, cnt_ref):
    i = pl.program_id(0)

    @pl.when(i == 0)
    def _init():
        out_ref[...] = jnp.zeros_like(out_ref)
        cnt_ref[...] = jnp.zeros_like(cnt_ref)

    hv = h_ref[...]
    sc = sc_ref[...]
    cl = cl_ref[...]
    onehot = (cl == lax.broadcasted_iota(jnp.int32, (BLK, KPAD), 1)).astype(jnp.float32)
    wrow = hv * sc
    out_ref[...] += lax.dot_general(onehot, wrow, (((0,), (0,)), ((), ())),
                                    preferred_element_type=jnp.float32)
    cnt_ref[...] += lax.dot_general(onehot, jnp.ones((BLK, 128), jnp.float32),
                                    (((0,), (0,)), ((), ())),
                                    preferred_element_type=jnp.float32)

    @pl.when(i == pl.num_programs(0) - 1)
    def _fin():
        out_ref[...] = out_ref[...] / jnp.maximum(cnt_ref[...][:, :1], 1.0)


@functools.partial(jax.jit, static_argnums=())
def _pool_pallas(h, score, cluster):
    hp = jnp.zeros((NPAD, D), jnp.float32).at[:N].set(h)
    sp = jnp.zeros((NPAD, 1), jnp.float32).at[:N, 0].set(score)
    cp = jnp.full((NPAD, 1), KPAD - 1, jnp.int32).at[:N, 0].set(cluster)
    grid = NPAD // BLK
    out, _ = pl.pallas_call(
        _pool_body,
        grid=(grid,),
        in_specs=[
            pl.BlockSpec((BLK, D), lambda i: (i, 0)),
            pl.BlockSpec((BLK, 1), lambda i: (i, 0)),
            pl.BlockSpec((BLK, 1), lambda i: (i, 0)),
        ],
        out_specs=[
            pl.BlockSpec((KPAD, D), lambda i: (0, 0)),
            pl.BlockSpec((KPAD, 128), lambda i: (0, 0)),
        ],
        out_shape=[
            jax.ShapeDtypeStruct((KPAD, D), jnp.float32),
            jax.ShapeDtypeStruct((KPAD, 128), jnp.float32),
        ],
    )(hp, sp, cp)
    return out[:K]


def kernel(x, edge_index, node_type, data_id, tree, x_y_index, batch,
           ln_gamma, ln_beta, W_conv, bias_conv, att_src_intra, att_dst_intra,
           att_src_inter, att_dst_inter, pool_w):
    xy = x_y_index * 2.0 - 1.0
    h = _ln(x, ln_gamma, ln_beta)
    h = _conv(h, edge_index, node_type, W_conv, bias_conv,
              att_src_intra, att_dst_intra, att_src_inter, att_dst_inter)
    h = _ln(h, ln_gamma, ln_beta)

    w = pool_w
    score = jnp.tanh(h @ (w / (jnp.linalg.norm(w) + 1e-12)))
    _, kept = jax.lax.top_k(score, K)
    centers = xy[kept]
    d2 = (jnp.sum(xy * xy, axis=-1)[:, None]
          + jnp.sum(centers * centers, axis=-1)[None, :]
          - 2.0 * (xy @ centers.T))
    cluster = jnp.argmin(d2, axis=1)

    new_x = _pool_pallas(h, score, cluster)
    new_edge = _sc_gather(cluster.astype(jnp.int32),
                          edge_index.reshape(2 * E)).reshape(2, E)
    nt = node_type[kept]
    tr = tree[kept]
    return (new_x, new_edge.astype(jnp.int32), nt, tr)


# speedup vs baseline: 9.5552x; 1.0378x over previous
import functools

import jax
import jax.numpy as jnp
from jax import lax
from jax.experimental import pallas as pl
from jax.experimental.pallas import tpu as pltpu
from jax.experimental.pallas import tpu_sc as plsc

N = 10000
E = 320000
D = 128
K = 1000
KPAD = 1024
NPAD = 10240
BLK = 1280


_NW = 32


def _sc_gather(table, idx):
    B = idx.shape[0]
    bpw = B // _NW
    mesh = plsc.VectorSubcoreMesh(core_axis_name="c", subcore_axis_name="s")

    @functools.partial(
        pl.kernel, mesh=mesh,
        out_type=jax.ShapeDtypeStruct((B,), table.dtype),
        scratch_types=[
            pltpu.VMEM((bpw,), jnp.int32),
            pltpu.VMEM((bpw,), table.dtype),
            pltpu.SemaphoreType.DMA,
        ],
    )
    def k(table_hbm, idx_hbm, out_hbm, idx_v, rows_v, sem):
        wid = lax.axis_index("s") * 2 + lax.axis_index("c")
        base = wid * bpw
        pltpu.sync_copy(idx_hbm.at[pl.ds(base, bpw)], idx_v)
        pltpu.async_copy(table_hbm.at[idx_v], rows_v, sem).wait()
        pltpu.sync_copy(rows_v, out_hbm.at[pl.ds(base, bpw)])

    return k(table, idx)


def _sc_gather_rows(table, idx):
    B = idx.shape[0]
    bpw = B // _NW
    CH = 400
    nch = bpw // CH
    mesh = plsc.VectorSubcoreMesh(core_axis_name="c", subcore_axis_name="s")

    @functools.partial(
        pl.kernel, mesh=mesh,
        out_type=jax.ShapeDtypeStruct((B, D), table.dtype),
        scratch_types=[
            pltpu.VMEM((bpw,), jnp.int32),
            pltpu.VMEM((CH, D), table.dtype),
            pltpu.SemaphoreType.DMA,
        ],
    )
    def k(table_hbm, idx_hbm, out_hbm, idx_v, rows_v, sem):
        wid = lax.axis_index("s") * 2 + lax.axis_index("c")
        base = wid * bpw
        pltpu.sync_copy(idx_hbm.at[pl.ds(base, bpw)], idx_v)
        for j in range(nch):
            pltpu.async_copy(table_hbm.at[idx_v.at[pl.ds(j * CH, CH)]],
                             rows_v, sem).wait()
            pltpu.sync_copy(rows_v, out_hbm.at[pl.ds(base + j * CH, CH)])

    return k(table, idx)


def _sc_gather6(t_si, t_di, t_se, t_de, t_nt, src, dst):
    B = src.shape[0]
    bpw = B // _NW
    mesh = plsc.VectorSubcoreMesh(core_axis_name="c", subcore_axis_name="s")

    @functools.partial(
        pl.kernel, mesh=mesh,
        out_type=[jax.ShapeDtypeStruct((B,), jnp.float32)] * 4
                 + [jax.ShapeDtypeStruct((B,), jnp.int32)] * 2,
        scratch_types=[
            pltpu.VMEM((bpw,), jnp.int32),
            pltpu.VMEM((bpw,), jnp.int32),
            pltpu.VMEM((bpw,), jnp.float32),
            pltpu.VMEM((bpw,), jnp.int32),
            pltpu.SemaphoreType.DMA,
        ],
    )
    def k(si_h, di_h, se_h, de_h, nt_h, src_h, dst_h,
          o_si, o_di, o_se, o_de, o_nts, o_ntd,
          src_v, dst_v, buf_f, buf_i, sem):
        wid = lax.axis_index("s") * 2 + lax.axis_index("c")
        base = wid * bpw
        pltpu.sync_copy(src_h.at[pl.ds(base, bpw)], src_v)
        pltpu.sync_copy(dst_h.at[pl.ds(base, bpw)], dst_v)
        for tab, idx_v, buf, out in ((si_h, src_v, buf_f, o_si),
                                     (di_h, dst_v, buf_f, o_di),
                                     (se_h, src_v, buf_f, o_se),
                                     (de_h, dst_v, buf_f, o_de),
                                     (nt_h, src_v, buf_i, o_nts),
                                     (nt_h, dst_v, buf_i, o_ntd)):
            pltpu.async_copy(tab.at[idx_v], buf, sem).wait()
            pltpu.sync_copy(buf, out.at[pl.ds(base, bpw)])

    return k(t_si, t_di, t_se, t_de, t_nt, src, dst)


def _sc_gather2(t1, t2, idx):
    B = idx.shape[0]
    bpw = B // _NW
    mesh = plsc.VectorSubcoreMesh(core_axis_name="c", subcore_axis_name="s")

    @functools.partial(
        pl.kernel, mesh=mesh,
        out_type=[jax.ShapeDtypeStruct((B,), t1.dtype),
                  jax.ShapeDtypeStruct((B,), t2.dtype)],
        scratch_types=[
            pltpu.VMEM((bpw,), jnp.int32),
            pltpu.VMEM((bpw,), t1.dtype),
            pltpu.SemaphoreType.DMA,
        ],
    )
    def k(t1_h, t2_h, idx_h, o1, o2, idx_v, buf, sem):
        wid = lax.axis_index("s") * 2 + lax.axis_index("c")
        base = wid * bpw
        pltpu.sync_copy(idx_h.at[pl.ds(base, bpw)], idx_v)
        for tab, out in ((t1_h, o1), (t2_h, o2)):
            pltpu.async_copy(tab.at[idx_v], buf, sem).wait()
            pltpu.sync_copy(buf, out.at[pl.ds(base, bpw)])

    return k(t1, t2, idx)


def _ln(x, g, b):
    m = jnp.mean(x, axis=-1, keepdims=True)
    v = jnp.var(x, axis=-1, keepdims=True)
    return (x - m) / jnp.sqrt(v + 1e-5) * g + b


def _dual_seg_softmax_agg(e_i, e_e, mask_i, mask_e, dst, ds, pi, h_src, n):
    maskf_i = mask_i.astype(jnp.float32)
    maskf_e = mask_e.astype(jnp.float32)
    e_i = jnp.where(mask_i, e_i, jnp.float32(-1e9))
    e_e = jnp.where(mask_e, e_e, jnp.float32(-1e9))
    eis, ees = _sc_gather2(e_i, e_e, pi)
    m_i = jax.ops.segment_max(eis, ds, num_segments=n, indices_are_sorted=True)
    m_e = jax.ops.segment_max(ees, ds, num_segments=n, indices_are_sorted=True)
    m_i = jnp.where(jnp.isfinite(m_i), m_i, jnp.float32(0.0))
    m_e = jnp.where(jnp.isfinite(m_e), m_e, jnp.float32(0.0))
    gmi, gme = _sc_gather2(m_i, m_e, dst)
    ex_i = jnp.exp(e_i - gmi) * maskf_i
    ex_e = jnp.exp(e_e - gme) * maskf_e
    den_i = jax.ops.segment_sum(ex_i, dst, num_segments=n)
    den_e = jax.ops.segment_sum(ex_e, dst, num_segments=n)
    gdi, gde = _sc_gather2(den_i, den_e, dst)
    al_i = ex_i / jnp.maximum(gdi, 1e-16)
    al_e = ex_e / jnp.maximum(gde, 1e-16)
    out = jax.ops.segment_sum(al_i[:, None] * h_src, dst, num_segments=n)
    return out + jax.ops.segment_sum(al_e[:, None] * h_src, dst, num_segments=n)


def _conv(x, edge_index, node_type, W, b, asi, adi, ase, ade):
    h = x @ W
    src = edge_index[0]
    dst = edge_index[1]
    ds, pi = lax.sort((dst, lax.iota(jnp.int32, dst.shape[0])), num_keys=1)
    n = x.shape[0]
    g_si, g_di, g_se, g_de, g_nts, g_ntd = _sc_gather6(
        h @ asi, h @ adi, h @ ase, h @ ade, node_type, src, dst)
    inter = g_nts != g_ntd
    e_intra = jax.nn.leaky_relu(g_si + g_di, 0.2)
    e_inter = jax.nn.leaky_relu(g_se + g_de, 0.2)
    h_src = _sc_gather_rows(h, src)
    out = _dual_seg_softmax_agg(e_intra, e_inter, jnp.logical_not(inter), inter,
                                dst, ds, pi, h_src, n)
    return out + b


def _pool_body(h_ref, sc_ref, cl_ref, out_ref, cnt_ref):
    i = pl.program_id(0)

    @pl.when(i == 0)
    def _init():
        out_ref[...] = jnp.zeros_like(out_ref)
        cnt_ref[...] = jnp.zeros_like(cnt_ref)

    hv = h_ref[...]
    sc = sc_ref[...]
    cl = cl_ref[...]
    onehot = (cl == lax.broadcasted_iota(jnp.int32, (BLK, KPAD), 1)).astype(jnp.float32)
    wrow = hv * sc
    out_ref[...] += lax.dot_general(onehot, wrow, (((0,), (0,)), ((), ())),
                                    preferred_element_type=jnp.float32)
    cnt_ref[...] += lax.dot_general(onehot, jnp.ones((BLK, 128), jnp.float32),
                                    (((0,), (0,)), ((), ())),
                                    preferred_element_type=jnp.float32)

    @pl.when(i == pl.num_programs(0) - 1)
    def _fin():
        out_ref[...] = out_ref[...] / jnp.maximum(cnt_ref[...][:, :1], 1.0)


@functools.partial(jax.jit, static_argnums=())
def _pool_pallas(h, score, cluster):
    hp = jnp.zeros((NPAD, D), jnp.float32).at[:N].set(h)
    sp = jnp.zeros((NPAD, 1), jnp.float32).at[:N, 0].set(score)
    cp = jnp.full((NPAD, 1), KPAD - 1, jnp.int32).at[:N, 0].set(cluster)
    grid = NPAD // BLK
    out, _ = pl.pallas_call(
        _pool_body,
        grid=(grid,),
        in_specs=[
            pl.BlockSpec((BLK, D), lambda i: (i, 0)),
            pl.BlockSpec((BLK, 1), lambda i: (i, 0)),
            pl.BlockSpec((BLK, 1), lambda i: (i, 0)),
        ],
        out_specs=[
            pl.BlockSpec((KPAD, D), lambda i: (0, 0)),
            pl.BlockSpec((KPAD, 128), lambda i: (0, 0)),
        ],
        out_shape=[
            jax.ShapeDtypeStruct((KPAD, D), jnp.float32),
            jax.ShapeDtypeStruct((KPAD, 128), jnp.float32),
        ],
    )(hp, sp, cp)
    return out[:K]


def kernel(x, edge_index, node_type, data_id, tree, x_y_index, batch,
           ln_gamma, ln_beta, W_conv, bias_conv, att_src_intra, att_dst_intra,
           att_src_inter, att_dst_inter, pool_w):
    xy = x_y_index * 2.0 - 1.0
    h = _ln(x, ln_gamma, ln_beta)
    h = _conv(h, edge_index, node_type, W_conv, bias_conv,
              att_src_intra, att_dst_intra, att_src_inter, att_dst_inter)
    h = _ln(h, ln_gamma, ln_beta)

    w = pool_w
    score = jnp.tanh(h @ (w / (jnp.linalg.norm(w) + 1e-12)))
    _, kept = jax.lax.top_k(score, K)
    centers = xy[kept]
    d2 = (jnp.sum(xy * xy, axis=-1)[:, None]
          + jnp.sum(centers * centers, axis=-1)[None, :]
          - 2.0 * (xy @ centers.T))
    cluster = jnp.argmin(d2, axis=1)

    new_x = _pool_pallas(h, score, cluster)
    new_edge = _sc_gather(cluster.astype(jnp.int32),
                          edge_index.reshape(2 * E)).reshape(2, E)
    nt = node_type[kept]
    tr = tree[kept]
    return (new_x, new_edge.astype(jnp.int32), nt, tr)


# speedup vs baseline: 9.5553x; 1.0000x over previous
import functools

import jax
import jax.numpy as jnp
from jax import lax
from jax.experimental import pallas as pl
from jax.experimental.pallas import tpu as pltpu
from jax.experimental.pallas import tpu_sc as plsc

N = 10000
E = 320000
D = 128
K = 1000
KPAD = 1024
NPAD = 10240
BLK = 1280


_NW = 32


def _sc_gather(table, idx):
    B = idx.shape[0]
    bpw = B // _NW
    mesh = plsc.VectorSubcoreMesh(core_axis_name="c", subcore_axis_name="s")

    @functools.partial(
        pl.kernel, mesh=mesh,
        out_type=jax.ShapeDtypeStruct((B,), table.dtype),
        scratch_types=[
            pltpu.VMEM((bpw,), jnp.int32),
            pltpu.VMEM((bpw,), table.dtype),
            pltpu.SemaphoreType.DMA,
        ],
    )
    def k(table_hbm, idx_hbm, out_hbm, idx_v, rows_v, sem):
        wid = lax.axis_index("s") * 2 + lax.axis_index("c")
        base = wid * bpw
        pltpu.sync_copy(idx_hbm.at[pl.ds(base, bpw)], idx_v)
        pltpu.async_copy(table_hbm.at[idx_v], rows_v, sem).wait()
        pltpu.sync_copy(rows_v, out_hbm.at[pl.ds(base, bpw)])

    return k(table, idx)


def _sc_gather_rows(table, idx):
    B = idx.shape[0]
    bpw = B // _NW
    CH = 400
    nch = bpw // CH
    mesh = plsc.VectorSubcoreMesh(core_axis_name="c", subcore_axis_name="s")

    @functools.partial(
        pl.kernel, mesh=mesh,
        out_type=jax.ShapeDtypeStruct((B, D), table.dtype),
        scratch_types=[
            pltpu.VMEM((bpw,), jnp.int32),
            pltpu.VMEM((CH, D), table.dtype),
            pltpu.VMEM((CH, D), table.dtype),
            pltpu.SemaphoreType.DMA,
            pltpu.SemaphoreType.DMA,
        ],
    )
    def k(table_hbm, idx_hbm, out_hbm, idx_v, rows_a, rows_b, sem_a, sem_b):
        wid = lax.axis_index("s") * 2 + lax.axis_index("c")
        base = wid * bpw
        pltpu.sync_copy(idx_hbm.at[pl.ds(base, bpw)], idx_v)
        bufs = (rows_a, rows_b)
        sems = (sem_a, sem_b)
        cps = [None, None]
        cps[0] = pltpu.async_copy(table_hbm.at[idx_v.at[pl.ds(0, CH)]],
                                  bufs[0], sems[0])
        for j in range(nch):
            p = j & 1
            cps[p].wait()
            if j + 1 < nch:
                q = (j + 1) & 1
                cps[q] = pltpu.async_copy(
                    table_hbm.at[idx_v.at[pl.ds((j + 1) * CH, CH)]],
                    bufs[q], sems[q])
            pltpu.sync_copy(bufs[p], out_hbm.at[pl.ds(base + j * CH, CH)])

    return k(table, idx)


def _sc_gather6(t_si, t_di, t_se, t_de, t_nt, src, dst):
    B = src.shape[0]
    bpw = B // _NW
    mesh = plsc.VectorSubcoreMesh(core_axis_name="c", subcore_axis_name="s")

    @functools.partial(
        pl.kernel, mesh=mesh,
        out_type=[jax.ShapeDtypeStruct((B,), jnp.float32)] * 4
                 + [jax.ShapeDtypeStruct((B,), jnp.int32)] * 2,
        scratch_types=[
            pltpu.VMEM((bpw,), jnp.int32),
            pltpu.VMEM((bpw,), jnp.int32),
            pltpu.VMEM((bpw,), jnp.float32),
            pltpu.VMEM((bpw,), jnp.int32),
            pltpu.SemaphoreType.DMA,
        ],
    )
    def k(si_h, di_h, se_h, de_h, nt_h, src_h, dst_h,
          o_si, o_di, o_se, o_de, o_nts, o_ntd,
          src_v, dst_v, buf_f, buf_i, sem):
        wid = lax.axis_index("s") * 2 + lax.axis_index("c")
        base = wid * bpw
        pltpu.sync_copy(src_h.at[pl.ds(base, bpw)], src_v)
        pltpu.sync_copy(dst_h.at[pl.ds(base, bpw)], dst_v)
        for tab, idx_v, buf, out in ((si_h, src_v, buf_f, o_si),
                                     (di_h, dst_v, buf_f, o_di),
                                     (se_h, src_v, buf_f, o_se),
                                     (de_h, dst_v, buf_f, o_de),
                                     (nt_h, src_v, buf_i, o_nts),
                                     (nt_h, dst_v, buf_i, o_ntd)):
            pltpu.async_copy(tab.at[idx_v], buf, sem).wait()
            pltpu.sync_copy(buf, out.at[pl.ds(base, bpw)])

    return k(t_si, t_di, t_se, t_de, t_nt, src, dst)


def _sc_gather2(t1, t2, idx):
    B = idx.shape[0]
    bpw = B // _NW
    mesh = plsc.VectorSubcoreMesh(core_axis_name="c", subcore_axis_name="s")

    @functools.partial(
        pl.kernel, mesh=mesh,
        out_type=[jax.ShapeDtypeStruct((B,), t1.dtype),
                  jax.ShapeDtypeStruct((B,), t2.dtype)],
        scratch_types=[
            pltpu.VMEM((bpw,), jnp.int32),
            pltpu.VMEM((bpw,), t1.dtype),
            pltpu.SemaphoreType.DMA,
        ],
    )
    def k(t1_h, t2_h, idx_h, o1, o2, idx_v, buf, sem):
        wid = lax.axis_index("s") * 2 + lax.axis_index("c")
        base = wid * bpw
        pltpu.sync_copy(idx_h.at[pl.ds(base, bpw)], idx_v)
        for tab, out in ((t1_h, o1), (t2_h, o2)):
            pltpu.async_copy(tab.at[idx_v], buf, sem).wait()
            pltpu.sync_copy(buf, out.at[pl.ds(base, bpw)])

    return k(t1, t2, idx)


def _ln(x, g, b):
    m = jnp.mean(x, axis=-1, keepdims=True)
    v = jnp.var(x, axis=-1, keepdims=True)
    return (x - m) / jnp.sqrt(v + 1e-5) * g + b


def _dual_seg_softmax_agg(e_i, e_e, mask_i, mask_e, dst, ds, pi, h_src, n):
    maskf_i = mask_i.astype(jnp.float32)
    maskf_e = mask_e.astype(jnp.float32)
    e_i = jnp.where(mask_i, e_i, jnp.float32(-1e9))
    e_e = jnp.where(mask_e, e_e, jnp.float32(-1e9))
    eis, ees = _sc_gather2(e_i, e_e, pi)
    m_i = jax.ops.segment_max(eis, ds, num_segments=n, indices_are_sorted=True)
    m_e = jax.ops.segment_max(ees, ds, num_segments=n, indices_are_sorted=True)
    m_i = jnp.where(jnp.isfinite(m_i), m_i, jnp.float32(0.0))
    m_e = jnp.where(jnp.isfinite(m_e), m_e, jnp.float32(0.0))
    gmi, gme = _sc_gather2(m_i, m_e, dst)
    ex_i = jnp.exp(e_i - gmi) * maskf_i
    ex_e = jnp.exp(e_e - gme) * maskf_e
    den_i = jax.ops.segment_sum(ex_i, dst, num_segments=n)
    den_e = jax.ops.segment_sum(ex_e, dst, num_segments=n)
    gdi, gde = _sc_gather2(den_i, den_e, dst)
    al_i = ex_i / jnp.maximum(gdi, 1e-16)
    al_e = ex_e / jnp.maximum(gde, 1e-16)
    out = jax.ops.segment_sum(al_i[:, None] * h_src, dst, num_segments=n)
    return out + jax.ops.segment_sum(al_e[:, None] * h_src, dst, num_segments=n)


def _conv(x, edge_index, node_type, W, b, asi, adi, ase, ade):
    h = x @ W
    src = edge_index[0]
    dst = edge_index[1]
    ds, pi = lax.sort((dst, lax.iota(jnp.int32, dst.shape[0])), num_keys=1)
    n = x.shape[0]
    g_si, g_di, g_se, g_de, g_nts, g_ntd = _sc_gather6(
        h @ asi, h @ adi, h @ ase, h @ ade, node_type, src, dst)
    inter = g_nts != g_ntd
    e_intra = jax.nn.leaky_relu(g_si + g_di, 0.2)
    e_inter = jax.nn.leaky_relu(g_se + g_de, 0.2)
    h_src = _sc_gather_rows(h, src)
    out = _dual_seg_softmax_agg(e_intra, e_inter, jnp.logical_not(inter), inter,
                                dst, ds, pi, h_src, n)
    return out + b


def _pool_body(h_ref, sc_ref, cl_ref, out_ref, cnt_ref):
    i = pl.program_id(0)

    @pl.when(i == 0)
    def _init():
        out_ref[...] = jnp.zeros_like(out_ref)
        cnt_ref[...] = jnp.zeros_like(cnt_ref)

    hv = h_ref[...]
    sc = sc_ref[...]
    cl = cl_ref[...]
    onehot = (cl == lax.broadcasted_iota(jnp.int32, (BLK, KPAD), 1)).astype(jnp.float32)
    wrow = hv * sc
    out_ref[...] += lax.dot_general(onehot, wrow, (((0,), (0,)), ((), ())),
                                    preferred_element_type=jnp.float32)
    cnt_ref[...] += lax.dot_general(onehot, jnp.ones((BLK, 128), jnp.float32),
                                    (((0,), (0,)), ((), ())),
                                    preferred_element_type=jnp.float32)

    @pl.when(i == pl.num_programs(0) - 1)
    def _fin():
        out_ref[...] = out_ref[...] / jnp.maximum(cnt_ref[...][:, :1], 1.0)


@functools.partial(jax.jit, static_argnums=())
def _pool_pallas(h, score, cluster):
    hp = jnp.zeros((NPAD, D), jnp.float32).at[:N].set(h)
    sp = jnp.zeros((NPAD, 1), jnp.float32).at[:N, 0].set(score)
    cp = jnp.full((NPAD, 1), KPAD - 1, jnp.int32).at[:N, 0].set(cluster)
    grid = NPAD // BLK
    out, _ = pl.pallas_call(
        _pool_body,
        grid=(grid,),
        in_specs=[
            pl.BlockSpec((BLK, D), lambda i: (i, 0)),
            pl.BlockSpec((BLK, 1), lambda i: (i, 0)),
            pl.BlockSpec((BLK, 1), lambda i: (i, 0)),
        ],
        out_specs=[
            pl.BlockSpec((KPAD, D), lambda i: (0, 0)),
            pl.BlockSpec((KPAD, 128), lambda i: (0, 0)),
        ],
        out_shape=[
            jax.ShapeDtypeStruct((KPAD, D), jnp.float32),
            jax.ShapeDtypeStruct((KPAD, 128), jnp.float32),
        ],
    )(hp, sp, cp)
    return out[:K]


def kernel(x, edge_index, node_type, data_id, tree, x_y_index, batch,
           ln_gamma, ln_beta, W_conv, bias_conv, att_src_intra, att_dst_intra,
           att_src_inter, att_dst_inter, pool_w):
    xy = x_y_index * 2.0 - 1.0
    h = _ln(x, ln_gamma, ln_beta)
    h = _conv(h, edge_index, node_type, W_conv, bias_conv,
              att_src_intra, att_dst_intra, att_src_inter, att_dst_inter)
    h = _ln(h, ln_gamma, ln_beta)

    w = pool_w
    score = jnp.tanh(h @ (w / (jnp.linalg.norm(w) + 1e-12)))
    _, kept = jax.lax.top_k(score, K)
    centers = xy[kept]
    d2 = (jnp.sum(xy * xy, axis=-1)[:, None]
          + jnp.sum(centers * centers, axis=-1)[None, :]
          - 2.0 * (xy @ centers.T))
    cluster = jnp.argmin(d2, axis=1)

    new_x = _pool_pallas(h, score, cluster)
    new_edge = _sc_gather(cluster.astype(jnp.int32),
                          edge_index.reshape(2 * E)).reshape(2, E)
    nt = node_type[kept]
    tr = tree[kept]
    return (new_x, new_edge.astype(jnp.int32), nt, tr)
